# baseline (speedup 1.0000x reference)
import functools

import numpy as np
import jax
import jax.numpy as jnp
from jax import lax
from jax.experimental import pallas as pl
from jax.experimental.pallas import tpu as pltpu
from jax.experimental.pallas import tpu_sc as plsc

B, U, T, D, H = 4, 2048, 4096, 256, 8
DH = D // H
K = 16
SCALE = 1.0 / np.sqrt(DH)

UB = 256
CW = 128
NCH = T // CW


def _topk_body(dist_ref, idx_ref):
    b = pl.program_id(0)
    f32 = jnp.float32
    INF = jnp.float32(np.inf)
    lane = lax.broadcasted_iota(jnp.int32, (UB, CW), 1).astype(f32)
    chunk_iota = lax.broadcasted_iota(jnp.int32, (UB, NCH), 1).astype(f32)
    ki = lax.broadcasted_iota(jnp.int32, (UB, K), 1).astype(f32)

    Ms, Gs = [], []
    for c in range(NCH):
        ch = dist_ref[0, :, c * CW:(c + 1) * CW]
        mc = jnp.min(ch, axis=1, keepdims=True)
        jc = jnp.min(jnp.where(ch == mc, lane, f32(CW)), axis=1, keepdims=True)
        Ms.append(mc)
        Gs.append(jc + c * CW)
    M0 = jnp.concatenate(Ms, axis=1)
    G0 = jnp.concatenate(Gs, axis=1)

    def round_(i, carry):
        M, G, out = carry
        m = jnp.min(M, axis=1, keepdims=True)
        j = jnp.min(jnp.where(M == m, G, f32(T)), axis=1, keepdims=True)
        out = jnp.where(ki == i.astype(f32), j, out)
        c_star = jnp.floor(j * (1.0 / CW))
        E = dist_ref[0, :, 0:CW]
        for c in range(1, NCH):
            E = jnp.where(c_star == c, dist_ref[0, :, c * CW:(c + 1) * CW], E)
        gl = c_star * CW + lane
        keep = (E > m) | ((E == m) & (gl > j))
        Em = jnp.where(keep, E, INF)
        newM = jnp.min(Em, axis=1, keepdims=True)
        newG = jnp.min(jnp.where(Em == newM, gl, f32(T)), axis=1, keepdims=True)
        M = jnp.where(chunk_iota == c_star, newM, M)
        G = jnp.where(chunk_iota == c_star, newG, G)
        return M, G, out

    _, _, out = lax.fori_loop(0, K, round_, (M0, G0, jnp.zeros((UB, K), f32)))
    idx_ref[0] = out.astype(jnp.int32) + b * T


def _topk(distances):
    nb = distances.shape[0]
    return pl.pallas_call(
        _topk_body,
        grid=(nb, U // UB),
        in_specs=[pl.BlockSpec((1, UB, T), lambda b, u: (b, u, 0))],
        out_specs=pl.BlockSpec((1, UB, K), lambda b, u: (b, u, 0)),
        out_shape=jax.ShapeDtypeStruct((nb, U, K), jnp.int32),
    )(distances)


_NC, _NS = 2, 16
_NW = _NC * _NS
_CH = 128


def _gather_body(table_hbm, idx_hbm, out_hbm, idx0, idx1, rows0, rows1,
                 isem0, isem1, gsem0, gsem1, osem0, osem1, *, _RPW, _NCHUNK):
    wid = lax.axis_index("s") * _NC + lax.axis_index("c")
    base = wid * _RPW
    idxs, rows = (idx0, idx1), (rows0, rows1)
    isems, gsems, osems = (isem0, isem1), (gsem0, gsem1), (osem0, osem1)

    def off_of(c):
        return pl.multiple_of(base + c * _CH, _CH)

    pltpu.async_copy(idx_hbm.at[pl.ds(off_of(0), _CH)], idxs[0], isems[0])
    pltpu.async_copy(idx_hbm.at[pl.ds(off_of(1), _CH)], idxs[1], isems[1])

    def pair(g, carry):
        for b in range(2):
            c = g * 2 + b
            off = off_of(c)

            @pl.when(g > 0)
            def _():
                pltpu.make_async_copy(
                    rows[b], out_hbm.at[pl.ds(off_of(c - 2), _CH)], osems[b]
                ).wait()

            pltpu.make_async_copy(
                idx_hbm.at[pl.ds(off, _CH)], idxs[b], isems[b]).wait()
            pltpu.async_copy(table_hbm.at[idxs[b]], rows[b], gsems[b])
            pltpu.make_async_copy(table_hbm.at[idxs[b]], rows[b], gsems[b]).wait()

            @pl.when(c + 2 < _NCHUNK)
            def _():
                pltpu.async_copy(
                    idx_hbm.at[pl.ds(off_of(c + 2), _CH)], idxs[b], isems[b])

            pltpu.async_copy(rows[b], out_hbm.at[pl.ds(off, _CH)], osems[b])
        return carry

    lax.fori_loop(0, _NCHUNK // 2, pair, 0)
    for b in range(2):
        pltpu.make_async_copy(
            rows[b],
            out_hbm.at[pl.ds(off_of(_NCHUNK - 2 + b), _CH)],
            osems[b],
        ).wait()


@functools.cache
def _make_gather(nrows):
    rpw = nrows // _NW
    nchunk = rpw // _CH
    body = functools.partial(_gather_body, _RPW=rpw, _NCHUNK=nchunk)
    return pl.kernel(
        body,
        out_type=jax.ShapeDtypeStruct((nrows, D), jnp.float32),
        mesh=plsc.VectorSubcoreMesh(core_axis_name="c", subcore_axis_name="s"),
        scratch_types=[
            pltpu.VMEM((_CH,), jnp.int32),
            pltpu.VMEM((_CH,), jnp.int32),
            pltpu.VMEM((_CH, D), jnp.float32),
            pltpu.VMEM((_CH, D), jnp.float32),
            pltpu.SemaphoreType.DMA,
            pltpu.SemaphoreType.DMA,
            pltpu.SemaphoreType.DMA,
            pltpu.SemaphoreType.DMA,
            pltpu.SemaphoreType.DMA,
            pltpu.SemaphoreType.DMA,
        ],
    )


QB = 128


def _attn_body(uav_ref, sel_ref, wq_ref, wk_ref, wv_ref, wo_ref, bo_ref, out_ref):
    f32 = jnp.float32
    bf16 = jnp.bfloat16
    dims_t = (((1,), (1,)), ((), ()))

    x = uav_ref[...].astype(bf16)
    q = lax.dot_general(x, wq_ref[...].astype(bf16), dims_t,
                        preferred_element_type=f32)
    sel = sel_ref[...].astype(bf16)
    kmat = lax.dot_general(sel, wk_ref[...].astype(bf16), dims_t,
                           preferred_element_type=f32)
    vmat = lax.dot_general(sel, wv_ref[...].astype(bf16), dims_t,
                           preferred_element_type=f32)

    dd = lax.broadcasted_iota(jnp.int32, (D, H), 0)
    hh = lax.broadcasted_iota(jnp.int32, (D, H), 1)
    G = (dd // DH == hh).astype(f32)

    qe = jnp.broadcast_to(q[:, None, :], (QB, K, D)).reshape(QB * K, D)
    prod = qe * kmat
    scores = lax.dot_general(prod, G, (((1,), (0,)), ((), ())),
                             preferred_element_type=f32) * SCALE

    s = scores.reshape(QB, K, H)
    m = jnp.max(s, axis=1, keepdims=True)
    e = jnp.exp(s - m)
    p = (e / jnp.sum(e, axis=1, keepdims=True)).reshape(QB * K, H)

    pfull = lax.dot_general(p, G, (((1,), (1,)), ((), ())),
                            preferred_element_type=f32)
    ctx = (pfull * vmat).reshape(QB, K, D)
    attn_out = jnp.sum(ctx, axis=1)

    out = lax.dot_general(attn_out.astype(bf16), wo_ref[...].astype(bf16),
                          dims_t, preferred_element_type=f32) + bo_ref[...]
    out_ref[...] = out


def _attention(uav_flat, sel, Wq, Wk, Wv, Wo, bo2):
    nq = uav_flat.shape[0]
    wspec = pl.BlockSpec((D, D), lambda i: (0, 0))
    return pl.pallas_call(
        _attn_body,
        grid=(nq // QB,),
        in_specs=[
            pl.BlockSpec((QB, D), lambda i: (i, 0)),
            pl.BlockSpec((QB * K, D), lambda i: (i, 0)),
            wspec, wspec, wspec, wspec,
            pl.BlockSpec((1, D), lambda i: (0, 0)),
        ],
        out_specs=pl.BlockSpec((QB, D), lambda i: (i, 0)),
        out_shape=jax.ShapeDtypeStruct((nq, D), jnp.float32),
    )(uav_flat, sel, Wq, Wk, Wv, Wo, bo2)


def kernel(uav_embeddings, target_embeddings, distances, Wq, Wk, Wv, Wo, bo):
    table = target_embeddings.reshape(B * T, D)
    bo2 = bo.reshape(1, D)
    hb = B // 2
    g = _make_gather(hb * U * K)
    idx0 = _topk(distances[:hb])
    idx1 = _topk(distances[hb:])
    sel0 = g(table, idx0.reshape(hb * U * K))
    sel1 = g(table, (idx1 + hb * T).reshape(hb * U * K))
    out0 = _attention(uav_embeddings[:hb].reshape(hb * U, D), sel0,
                      Wq, Wk, Wv, Wo, bo2)
    out1 = _attention(uav_embeddings[hb:].reshape(hb * U, D), sel1,
                      Wq, Wk, Wv, Wo, bo2)
    return jnp.concatenate([out0, out1], axis=0).reshape(B, U, D)


# speedup vs baseline: 1.0416x; 1.0416x over previous
import functools

import numpy as np
import jax
import jax.numpy as jnp
from jax import lax
from jax.experimental import pallas as pl
from jax.experimental.pallas import tpu as pltpu
from jax.experimental.pallas import tpu_sc as plsc

B, U, T, D, H = 4, 2048, 4096, 256, 8
DH = D // H
K = 16
SCALE = 1.0 / np.sqrt(DH)

UB = 256
CW = 128
NCH = T // CW


def _topk_body(dist_ref, idx_ref):
    b = pl.program_id(0)
    f32 = jnp.float32
    INF = jnp.float32(np.inf)
    lane = lax.broadcasted_iota(jnp.int32, (UB, CW), 1).astype(f32)
    chunk_iota = lax.broadcasted_iota(jnp.int32, (UB, NCH), 1).astype(f32)
    ki = lax.broadcasted_iota(jnp.int32, (UB, K), 1).astype(f32)

    Ms, Gs = [], []
    for c in range(NCH):
        ch = dist_ref[0, :, c * CW:(c + 1) * CW]
        mc = jnp.min(ch, axis=1, keepdims=True)
        jc = jnp.min(jnp.where(ch == mc, lane, f32(CW)), axis=1, keepdims=True)
        Ms.append(mc)
        Gs.append(jc + c * CW)
    M0 = jnp.concatenate(Ms, axis=1)
    G0 = jnp.concatenate(Gs, axis=1)

    def round_(i, carry):
        M, G, out = carry
        m = jnp.min(M, axis=1, keepdims=True)
        j = jnp.min(jnp.where(M == m, G, f32(T)), axis=1, keepdims=True)
        out = jnp.where(ki == i.astype(f32), j, out)
        c_star = jnp.floor(j * (1.0 / CW))
        E = dist_ref[0, :, 0:CW]
        for c in range(1, NCH):
            E = jnp.where(c_star == c, dist_ref[0, :, c * CW:(c + 1) * CW], E)
        gl = c_star * CW + lane
        keep = (E > m) | ((E == m) & (gl > j))
        Em = jnp.where(keep, E, INF)
        newM = jnp.min(Em, axis=1, keepdims=True)
        newG = jnp.min(jnp.where(Em == newM, gl, f32(T)), axis=1, keepdims=True)
        M = jnp.where(chunk_iota == c_star, newM, M)
        G = jnp.where(chunk_iota == c_star, newG, G)
        return M, G, out

    _, _, out = lax.fori_loop(0, K, round_, (M0, G0, jnp.zeros((UB, K), f32)))
    idx_ref[0] = out.astype(jnp.int32) + b * T


def _topk(distances):
    nb = distances.shape[0]
    return pl.pallas_call(
        _topk_body,
        grid=(nb, U // UB),
        in_specs=[pl.BlockSpec((1, UB, T), lambda b, u: (b, u, 0))],
        out_specs=pl.BlockSpec((1, UB, K), lambda b, u: (b, u, 0)),
        out_shape=jax.ShapeDtypeStruct((nb, U, K), jnp.int32),
    )(distances)


_NC, _NS = 2, 16
_NW = _NC * _NS
_CH = 128


def _gather_body(table_hbm, idx_hbm, out_hbm, idx0, idx1, rows0, rows1,
                 isem0, isem1, gsem0, gsem1, osem0, osem1, *, _RPW, _NCHUNK):
    wid = lax.axis_index("s") * _NC + lax.axis_index("c")
    base = wid * _RPW
    idxs, rows = (idx0, idx1), (rows0, rows1)
    isems, gsems, osems = (isem0, isem1), (gsem0, gsem1), (osem0, osem1)

    def off_of(c):
        return pl.multiple_of(base + c * _CH, _CH)

    pltpu.async_copy(idx_hbm.at[pl.ds(off_of(0), _CH)], idxs[0], isems[0])
    pltpu.async_copy(idx_hbm.at[pl.ds(off_of(1), _CH)], idxs[1], isems[1])

    def pair(g, carry):
        for b in range(2):
            c = g * 2 + b
            off = off_of(c)

            @pl.when(g > 0)
            def _():
                pltpu.make_async_copy(
                    rows[b], out_hbm.at[pl.ds(off_of(c - 2), _CH)], osems[b]
                ).wait()

            pltpu.make_async_copy(
                idx_hbm.at[pl.ds(off, _CH)], idxs[b], isems[b]).wait()
            pltpu.async_copy(table_hbm.at[idxs[b]], rows[b], gsems[b])
            pltpu.make_async_copy(table_hbm.at[idxs[b]], rows[b], gsems[b]).wait()

            @pl.when(c + 2 < _NCHUNK)
            def _():
                pltpu.async_copy(
                    idx_hbm.at[pl.ds(off_of(c + 2), _CH)], idxs[b], isems[b])

            pltpu.async_copy(rows[b], out_hbm.at[pl.ds(off, _CH)], osems[b])
        return carry

    lax.fori_loop(0, _NCHUNK // 2, pair, 0)
    for b in range(2):
        pltpu.make_async_copy(
            rows[b],
            out_hbm.at[pl.ds(off_of(_NCHUNK - 2 + b), _CH)],
            osems[b],
        ).wait()


@functools.cache
def _make_gather(nrows):
    rpw = nrows // _NW
    nchunk = rpw // _CH
    body = functools.partial(_gather_body, _RPW=rpw, _NCHUNK=nchunk)
    return pl.kernel(
        body,
        out_type=jax.ShapeDtypeStruct((nrows, D), jnp.float32),
        mesh=plsc.VectorSubcoreMesh(core_axis_name="c", subcore_axis_name="s"),
        scratch_types=[
            pltpu.VMEM((_CH,), jnp.int32),
            pltpu.VMEM((_CH,), jnp.int32),
            pltpu.VMEM((_CH, D), jnp.float32),
            pltpu.VMEM((_CH, D), jnp.float32),
            pltpu.SemaphoreType.DMA,
            pltpu.SemaphoreType.DMA,
            pltpu.SemaphoreType.DMA,
            pltpu.SemaphoreType.DMA,
            pltpu.SemaphoreType.DMA,
            pltpu.SemaphoreType.DMA,
        ],
    )


QB = 128


def _attn_body(uav_ref, sel_ref, wq_ref, wk_ref, wv_ref, wo_ref, bo_ref, out_ref):
    f32 = jnp.float32
    bf16 = jnp.bfloat16
    dims_t = (((1,), (1,)), ((), ()))

    x = uav_ref[...].astype(bf16)
    q = lax.dot_general(x, wq_ref[...].astype(bf16), dims_t,
                        preferred_element_type=f32)
    sel = sel_ref[...].astype(bf16)
    kmat = lax.dot_general(sel, wk_ref[...].astype(bf16), dims_t,
                           preferred_element_type=f32)
    vmat = lax.dot_general(sel, wv_ref[...].astype(bf16), dims_t,
                           preferred_element_type=f32)

    dd = lax.broadcasted_iota(jnp.int32, (D, H), 0)
    hh = lax.broadcasted_iota(jnp.int32, (D, H), 1)
    G = (dd // DH == hh).astype(f32)

    qe = jnp.broadcast_to(q[:, None, :], (QB, K, D)).reshape(QB * K, D)
    prod = qe * kmat
    scores = lax.dot_general(prod, G, (((1,), (0,)), ((), ())),
                             preferred_element_type=f32) * SCALE

    s = scores.reshape(QB, K, H)
    m = jnp.max(s, axis=1, keepdims=True)
    e = jnp.exp(s - m)
    p = (e / jnp.sum(e, axis=1, keepdims=True)).reshape(QB * K, H)

    pfull = lax.dot_general(p, G, (((1,), (1,)), ((), ())),
                            preferred_element_type=f32)
    ctx = (pfull * vmat).reshape(QB, K, D)
    attn_out = jnp.sum(ctx, axis=1)

    out = lax.dot_general(attn_out.astype(bf16), wo_ref[...].astype(bf16),
                          dims_t, preferred_element_type=f32) + bo_ref[...]
    out_ref[...] = out


def _attention(uav_flat, sel, Wq, Wk, Wv, Wo, bo2):
    nq = uav_flat.shape[0]
    wspec = pl.BlockSpec((D, D), lambda i: (0, 0))
    return pl.pallas_call(
        _attn_body,
        grid=(nq // QB,),
        in_specs=[
            pl.BlockSpec((QB, D), lambda i: (i, 0)),
            pl.BlockSpec((QB * K, D), lambda i: (i, 0)),
            wspec, wspec, wspec, wspec,
            pl.BlockSpec((1, D), lambda i: (0, 0)),
        ],
        out_specs=pl.BlockSpec((QB, D), lambda i: (i, 0)),
        out_shape=jax.ShapeDtypeStruct((nq, D), jnp.float32),
    )(uav_flat, sel, Wq, Wk, Wv, Wo, bo2)


def kernel(uav_embeddings, target_embeddings, distances, Wq, Wk, Wv, Wo, bo):
    idx = _topk(distances)
    sel = _make_gather(B * U * K)(target_embeddings.reshape(B * T, D),
                                  idx.reshape(B * U * K))
    out = _attention(uav_embeddings.reshape(B * U, D), sel,
                     Wq, Wk, Wv, Wo, bo.reshape(1, D))
    return out.reshape(B, U, D)
